# baseline (speedup 1.0000x reference)
import functools

import jax
import jax.numpy as jnp
from jax import lax
from jax.experimental import pallas as pl
from jax.experimental.pallas import tpu as pltpu
from jax.experimental.pallas import tpu_sc as plsc

_B = 16384
_K = 32
_D = 26
_NC = 2
_NS = 16
_NW = _NC * _NS
_BPW = _B // _NW
_NCH = _BPW // 128


def _sc_gather(ue, be, ul16, bl16, uidx, bidx, uhi, bhi):
    mesh = plsc.VectorSubcoreMesh(
        core_axis_name="c", subcore_axis_name="s",
        num_cores=_NC, num_subcores=_NS)

    @functools.partial(
        pl.kernel,
        out_type=(
            jax.ShapeDtypeStruct((_B, _K), jnp.float32),
            jax.ShapeDtypeStruct((_B, _K), jnp.float32),
            jax.ShapeDtypeStruct((_B, 16), jnp.float32),
            jax.ShapeDtypeStruct((_B, 16), jnp.float32),
        ),
        mesh=mesh,
        compiler_params=pltpu.CompilerParams(use_tc_tiling_on_sc=False),
        scratch_types=[
            pltpu.VMEM((_NCH, 128), jnp.int32),
            pltpu.VMEM((_NCH, 128), jnp.int32),
            pltpu.VMEM((_NCH, 128), jnp.int32),
            pltpu.VMEM((_NCH, 128), jnp.int32),
            pltpu.VMEM((_BPW, _K), jnp.float32),
            pltpu.VMEM((_BPW, _K), jnp.float32),
            pltpu.VMEM((_BPW, 16), jnp.float32),
            pltpu.VMEM((_BPW, 16), jnp.float32),
            pltpu.SemaphoreType.DMA,
        ],
    )
    def k(ue_hbm, be_hbm, ul_hbm, bl_hbm, uidx_hbm, bidx_hbm,
          uhi_hbm, bhi_hbm,
          uo_hbm, bo_hbm, ulo_hbm, blo_hbm,
          uidx_v, bidx_v, uhi_v, bhi_v, urows_v, brows_v, ulv, blv, sem):
        wid = lax.axis_index("s") * _NC + lax.axis_index("c")
        cbase = wid * _NCH
        pltpu.sync_copy(uidx_hbm.at[pl.ds(cbase, _NCH)], uidx_v)
        pltpu.sync_copy(bidx_hbm.at[pl.ds(cbase, _NCH)], bidx_v)
        pltpu.sync_copy(uhi_hbm.at[pl.ds(cbase, _NCH)], uhi_v)
        pltpu.sync_copy(bhi_hbm.at[pl.ds(cbase, _NCH)], bhi_v)
        copies = []
        for j in range(_NCH):
            sl = pl.ds(j * 128, 128)
            copies.append(pltpu.async_copy(
                ue_hbm.at[uidx_v.at[j]], urows_v.at[sl], sem))
            copies.append(pltpu.async_copy(
                be_hbm.at[bidx_v.at[j]], brows_v.at[sl], sem))
            copies.append(pltpu.async_copy(
                ul_hbm.at[uhi_v.at[j]], ulv.at[sl], sem))
            copies.append(pltpu.async_copy(
                bl_hbm.at[bhi_v.at[j]], blv.at[sl], sem))
        for c in copies:
            c.wait()
        base = wid * _BPW
        pltpu.sync_copy(urows_v, uo_hbm.at[pl.ds(base, _BPW)])
        pltpu.sync_copy(brows_v, bo_hbm.at[pl.ds(base, _BPW)])
        pltpu.sync_copy(ulv, ulo_hbm.at[pl.ds(base, _BPW)])
        pltpu.sync_copy(blv, blo_hbm.at[pl.ds(base, _BPW)])

    return k(ue, be, ul16, bl16, uidx, bidx, uhi, bhi)


def _tc_body(u_ref, b_ref, ul16_ref, bl16_ref, ulane_ref, blane_ref,
             num_ref, bias_ref, nl_ref, ne_ref, g_ref, bt_ref, out_ref):
    eps = 1e-5
    num_n = num_ref[...] * (g_ref[...] * lax.rsqrt(1.0 + eps)) + bt_ref[...]
    ne = ne_ref[...]
    nsum = jnp.dot(num_n, ne, preferred_element_type=jnp.float32)
    u = u_ref[...]
    b = b_ref[...]
    s = u + b + nsum
    e2 = jnp.sum(ne * ne, axis=1, keepdims=True)
    nsq = jnp.dot(num_n * num_n, e2, preferred_element_type=jnp.float32)
    lanes = lax.broadcasted_iota(jnp.int32, ul16_ref.shape, 1)
    ul = jnp.sum(jnp.where(lanes == ulane_ref[...], ul16_ref[...], 0.0),
                 axis=1, keepdims=True)
    bl = jnp.sum(jnp.where(lanes == blane_ref[...], bl16_ref[...], 0.0),
                 axis=1, keepdims=True)
    lin = (bias_ref[...] + ul + bl
           + jnp.dot(num_n, nl_ref[...], preferred_element_type=jnp.float32))
    inter = 0.5 * (jnp.sum(s * s, axis=1, keepdims=True)
                   - jnp.sum(u * u, axis=1, keepdims=True)
                   - jnp.sum(b * b, axis=1, keepdims=True)
                   - nsq)
    out_ref[...] = jnp.clip(lin + inter, 1.0, 5.0)


def _tc_combine(urows, brows, ul16, bl16, ulane, blane, num, bias,
                num_lin, num_emb, bn_gamma, bn_beta):
    blk = 2048
    grid = (_B // blk,)
    row = lambda i: (i, 0)
    rep = lambda i: (0, 0)
    return pl.pallas_call(
        _tc_body,
        grid=grid,
        in_specs=[
            pl.BlockSpec((blk, _K), row),
            pl.BlockSpec((blk, _K), row),
            pl.BlockSpec((blk, 16), row),
            pl.BlockSpec((blk, 16), row),
            pl.BlockSpec((blk, 1), row),
            pl.BlockSpec((blk, 1), row),
            pl.BlockSpec((blk, _D), row),
            pl.BlockSpec((1, 1), rep),
            pl.BlockSpec((_D, 1), rep),
            pl.BlockSpec((_D, _K), rep),
            pl.BlockSpec((1, _D), rep),
            pl.BlockSpec((1, _D), rep),
        ],
        out_specs=pl.BlockSpec((blk, 1), row),
        out_shape=jax.ShapeDtypeStruct((_B, 1), jnp.float32),
    )(urows, brows, ul16, bl16, ulane, blane, num, bias, num_lin,
      num_emb, bn_gamma, bn_beta)


def kernel(user, biz, num, bias, user_lin_w, biz_lin_w, num_lin,
           user_emb_w, biz_emb_w, num_emb, bn_gamma, bn_beta):
    user = user.astype(jnp.int32)
    biz = biz.astype(jnp.int32)
    uidx = user.reshape(_B // 128, 128)
    bidx = biz.reshape(_B // 128, 128)
    uhi = (user >> 4).reshape(_B // 128, 128)
    bhi = (biz >> 4).reshape(_B // 128, 128)
    ul16 = user_lin_w.reshape(-1, 16)
    bl16 = biz_lin_w.reshape(-1, 16)
    urows, brows, ulg, blg = _sc_gather(
        user_emb_w, biz_emb_w, ul16, bl16, uidx, bidx, uhi, bhi)
    out = _tc_combine(
        urows, brows, ulg, blg,
        (user & 15).reshape(_B, 1), (biz & 15).reshape(_B, 1), num,
        bias.reshape(1, 1), num_lin.reshape(_D, 1), num_emb,
        bn_gamma.reshape(1, _D), bn_beta.reshape(1, _D))
    return out.reshape(_B)
